# Initial kernel scaffold; baseline (speedup 1.0000x reference)
import jax
import jax.numpy as jnp
from jax.experimental import pallas as pl


def kernel(predictions, user_id, item_id, u):
    raise NotImplementedError("write your pallas kernel here")


# speedup vs baseline: 3.8270x; 3.8270x over previous
import functools

import jax
import jax.numpy as jnp
from jax import lax
from jax.experimental import pallas as pl
from jax.experimental.pallas import tpu as pltpu
from jax.experimental.pallas import tpu_sc as plsc

_NUM_POS = 10
_GAMMA = 0.9
_EPS = 1e-10
_N_NEG = 1000
_NCOL = _NUM_POS + _N_NEG
_B = 4096
_K = _B * _NUM_POS
_UROWS = 50001
_UCOLS = 1001
_TBL = _UROWS * _UCOLS
_RB = 256
_NBLK = _B // _RB

_NTILES = 32
_KPT = _K // _NTILES
_CH = 128
_NCH = _KPT // _CH
_NV = _KPT // 16


def _tc_body(pred_ref, uid_ref, iid_ref, vpre_ref, numer_ref, fidx_ref,
             c_sc, s_sc, t_sc, pos_sc, mmax_sc):
    i = pl.program_id(0)

    @pl.when(i < _NBLK)
    def _row_stats():
        x = pred_ref[...]
        lane = lax.broadcasted_iota(jnp.int32, x.shape, 1)
        isneg = lane >= _NUM_POS
        c = jnp.max(jnp.where(isneg, x, -jnp.inf), axis=1, keepdims=True)
        e = jnp.where(isneg, jnp.exp(x - c), 0.0)
        s = jnp.sum(e, axis=1, keepdims=True)
        t = jnp.sum(x * e, axis=1, keepdims=True)
        m = jnp.min(jnp.where(isneg, jnp.inf, x), axis=1, keepdims=True)
        r0 = i * _RB
        c_sc[pl.ds(r0, _RB), :] = c
        s_sc[pl.ds(r0, _RB), :] = s
        t_sc[pl.ds(r0, _RB), :] = t
        pos_sc[pl.ds(r0, _RB), :] = x[:, :_NUM_POS]
        bm = jnp.max(c - m)

        @pl.when(i == 0)
        def _():
            mmax_sc[0, 0] = bm

        @pl.when(i > 0)
        def _():
            mmax_sc[0, 0] = jnp.maximum(mmax_sc[0, 0], bm)

    @pl.when(i == _NBLK)
    def _per_slot():
        c = c_sc[...]
        s = s_sc[...]
        t = t_sc[...]
        pos = pos_sc[...]
        big_m = mmax_sc[0, 0]
        e = jnp.exp(c - pos - big_m)
        vpre_ref[...] = (_GAMMA / _N_NEG) * s * e
        numer_ref[...] = e * (t - pos * s)
        fidx_ref[...] = uid_ref[...] * _UCOLS + iid_ref[...]


_tc_call = pl.pallas_call(
    _tc_body,
    grid=(_NBLK + 1,),
    in_specs=[
        pl.BlockSpec((_RB, _NCOL), lambda i: (jnp.minimum(i, _NBLK - 1), 0)),
        pl.BlockSpec((_B, 1), lambda i: (0, 0)),
        pl.BlockSpec((_B, _NUM_POS), lambda i: (0, 0)),
    ],
    out_specs=[
        pl.BlockSpec((_B, _NUM_POS), lambda i: (0, 0)),
        pl.BlockSpec((_B, _NUM_POS), lambda i: (0, 0)),
        pl.BlockSpec((_B, _NUM_POS), lambda i: (0, 0)),
    ],
    out_shape=[
        jax.ShapeDtypeStruct((_B, _NUM_POS), jnp.float32),
        jax.ShapeDtypeStruct((_B, _NUM_POS), jnp.float32),
        jax.ShapeDtypeStruct((_B, _NUM_POS), jnp.int32),
    ],
    scratch_shapes=[
        pltpu.VMEM((_B, 1), jnp.float32),
        pltpu.VMEM((_B, 1), jnp.float32),
        pltpu.VMEM((_B, 1), jnp.float32),
        pltpu.VMEM((_B, _NUM_POS), jnp.float32),
        pltpu.SMEM((1, 1), jnp.float32),
    ],
)


def _sc_scatter_body(fidx_hbm, vpre_hbm, u_hbm, tbl_hbm,
                     idxv, valv, uoldv, sem):
    wid = lax.axis_index("s") * 2 + lax.axis_index("c")
    k0 = wid * _KPT
    cps = [pltpu.async_copy(fidx_hbm.at[pl.ds(k0 + j * _CH, _CH)],
                            idxv.at[j], sem)
           for j in range(_NCH)]
    cps.append(pltpu.async_copy(vpre_hbm.at[pl.ds(k0, _KPT)], valv, sem))
    for cp in cps:
        cp.wait()
    cps = [pltpu.async_copy(u_hbm.at[idxv.at[j]],
                            uoldv.at[pl.ds(j * _CH, _CH)], sem)
           for j in range(_NCH)]
    for cp in cps:
        cp.wait()
    for v in range(_NV):
        sl = pl.ds(v * 16, 16)
        valv[sl] = valv[sl] + (1.0 - _GAMMA) * uoldv[sl]
    cps = [pltpu.async_copy(valv.at[pl.ds(j * _CH, _CH)],
                            tbl_hbm.at[idxv.at[j]], sem)
           for j in range(_NCH)]
    for cp in cps:
        cp.wait()


_sc_scatter = functools.partial(
    pl.kernel,
    mesh=plsc.VectorSubcoreMesh(core_axis_name="c", subcore_axis_name="s"),
    out_type=jax.ShapeDtypeStruct((_TBL,), jnp.float32),
    scratch_types=[
        pltpu.VMEM((_NCH, _CH), jnp.int32),
        pltpu.VMEM((_KPT,), jnp.float32),
        pltpu.VMEM((_KPT,), jnp.float32),
        pltpu.SemaphoreType.DMA,
    ],
)(_sc_scatter_body)


def _sc_reduce_body(fidx_hbm, numer_hbm, tbl_hbm, part_hbm,
                    idxv, numv, denv, accv, sem):
    wid = lax.axis_index("s") * 2 + lax.axis_index("c")
    k0 = wid * _KPT
    cps = [pltpu.async_copy(fidx_hbm.at[pl.ds(k0 + j * _CH, _CH)],
                            idxv.at[j], sem)
           for j in range(_NCH)]
    cps.append(pltpu.async_copy(numer_hbm.at[pl.ds(k0, _KPT)], numv, sem))
    for cp in cps:
        cp.wait()
    cps = [pltpu.async_copy(tbl_hbm.at[idxv.at[j]],
                            denv.at[pl.ds(j * _CH, _CH)], sem)
           for j in range(_NCH)]
    for cp in cps:
        cp.wait()
    acc = jnp.zeros((16,), jnp.float32)
    for v in range(_NV):
        sl = pl.ds(v * 16, 16)
        acc = acc + numv[sl] / (denv[sl] + _EPS)
    accv[...] = acc
    pltpu.sync_copy(accv, part_hbm.at[wid])


_sc_reduce = functools.partial(
    pl.kernel,
    mesh=plsc.VectorSubcoreMesh(core_axis_name="c", subcore_axis_name="s"),
    out_type=jax.ShapeDtypeStruct((_NTILES, 16), jnp.float32),
    scratch_types=[
        pltpu.VMEM((_NCH, _CH), jnp.int32),
        pltpu.VMEM((_KPT,), jnp.float32),
        pltpu.VMEM((_KPT,), jnp.float32),
        pltpu.VMEM((16,), jnp.float32),
        pltpu.SemaphoreType.DMA,
    ],
)(_sc_reduce_body)


def kernel(predictions, user_id, item_id, u):
    vpre, numer, fidx = _tc_call(
        predictions, user_id.reshape(_B, 1), item_id)
    fidx = fidx.reshape(_K)
    tbl = _sc_scatter(fidx, vpre.reshape(_K), u.reshape(_TBL))
    parts = _sc_reduce(fidx, numer.reshape(_K), tbl)
    return jnp.sum(parts) / _B
